# baseline (speedup 1.0000x reference)
import functools

import jax
import jax.numpy as jnp
from jax.experimental import pallas as pl
from jax.experimental.pallas import tpu as pltpu


def _se_step(x_ref, w1t_ref, b1_ref, w2t_ref, b2_ref, o_ref, *, inv_hw):
    x = x_ref[...]
    s = jnp.sum(x, axis=(2, 3), dtype=jnp.float32) * jnp.float32(inv_hw)
    h = jnp.dot(s, w1t_ref[...], preferred_element_type=jnp.float32) + b1_ref[...]
    h = h * jax.nn.sigmoid(h)
    g = jnp.dot(h, w2t_ref[...], preferred_element_type=jnp.float32) + b2_ref[...]
    g = jax.nn.sigmoid(g)
    o_ref[...] = x * g[:, :, None, None]


def kernel(x, w1, b1, w2, b2):
    N, C, H, W = x.shape
    R = w1.shape[0]

    itemsize = jnp.dtype(x.dtype).itemsize
    w_pad = max(W, 128) if W < 128 else W
    per_sample = C * H * w_pad * itemsize
    nb = 1
    for d in range(1, N + 1):
        if N % d == 0 and N // d >= 4 and 4 * d * per_sample <= (48 << 20):
            nb = d

    out = pl.pallas_call(
        functools.partial(_se_step, inv_hw=1.0 / (H * W)),
        out_shape=jax.ShapeDtypeStruct((N, C, H, W), x.dtype),
        grid=(N // nb,),
        in_specs=[
            pl.BlockSpec((nb, C, H, W), lambda i: (i, 0, 0, 0)),
            pl.BlockSpec((C, R), lambda i: (0, 0)),
            pl.BlockSpec((1, R), lambda i: (0, 0)),
            pl.BlockSpec((R, C), lambda i: (0, 0)),
            pl.BlockSpec((1, C), lambda i: (0, 0)),
        ],
        out_specs=pl.BlockSpec((nb, C, H, W), lambda i: (i, 0, 0, 0)),
        compiler_params=pltpu.CompilerParams(
            dimension_semantics=("parallel",),
            vmem_limit_bytes=int(56 << 20)),
    )(x,
      w1.T.astype(jnp.float32),
      b1.reshape(1, R).astype(jnp.float32),
      w2.T.astype(jnp.float32),
      b2.reshape(1, C).astype(jnp.float32))
    return out


# speedup vs baseline: 3.2092x; 3.2092x over previous
import functools

import jax
import jax.numpy as jnp
from jax.experimental import pallas as pl
from jax.experimental.pallas import tpu as pltpu

_LANE = 128


def _se_step(x_ref, w1_ref, b1_ref, w2_ref, b2_ref, o_ref, *, inv_hw):
    x = x_ref[...]
    s = jnp.sum(x, axis=(2, 3), dtype=jnp.float32) * jnp.float32(inv_hw)
    h = jax.lax.dot_general(s, w1_ref[...], (((1,), (1,)), ((), ())),
                            preferred_element_type=jnp.float32) + b1_ref[...]
    h = h * jax.nn.sigmoid(h)
    g = jax.lax.dot_general(h, w2_ref[...], (((1,), (1,)), ((), ())),
                            preferred_element_type=jnp.float32) + b2_ref[...]
    g = jax.nn.sigmoid(g)
    o_ref[...] = x * g[:, :, None, None]


def kernel(x, w1, b1, w2, b2):
    N, C, H, W = x.shape
    R = w1.shape[0]
    HW = H * W
    HWp = ((HW + _LANE - 1) // _LANE) * _LANE

    x_flat = x.reshape(N, C, HW)
    if HWp != HW:
        x_flat = jnp.pad(x_flat, ((0, 0), (0, 0), (0, HWp - HW)))
    rows = HWp // _LANE
    xv = x_flat.reshape(N, C, rows, _LANE)

    per_sample = C * HWp * jnp.dtype(x.dtype).itemsize
    nb = 1
    for d in range(1, N + 1):
        if N % d == 0 and N // d >= 4 and 4 * d * per_sample <= (48 << 20):
            nb = d

    out = pl.pallas_call(
        functools.partial(_se_step, inv_hw=1.0 / HW),
        out_shape=jax.ShapeDtypeStruct((N, C, rows, _LANE), x.dtype),
        grid=(N // nb,),
        in_specs=[
            pl.BlockSpec((nb, C, rows, _LANE), lambda i: (i, 0, 0, 0)),
            pl.BlockSpec((R, C), lambda i: (0, 0)),
            pl.BlockSpec((1, R), lambda i: (0, 0)),
            pl.BlockSpec((C, R), lambda i: (0, 0)),
            pl.BlockSpec((1, C), lambda i: (0, 0)),
        ],
        out_specs=pl.BlockSpec((nb, C, rows, _LANE), lambda i: (i, 0, 0, 0)),
        compiler_params=pltpu.CompilerParams(
            dimension_semantics=("parallel",),
            vmem_limit_bytes=int(56 << 20)),
    )(xv, w1, b1.reshape(1, R), w2, b2.reshape(1, C))

    out = out.reshape(N, C, HWp)
    if HWp != HW:
        out = out[:, :, :HW]
    return out.reshape(N, C, H, W)


# speedup vs baseline: 13.2125x; 4.1171x over previous
import functools

import jax
import jax.numpy as jnp
from jax.experimental import pallas as pl
from jax.experimental.pallas import tpu as pltpu


def _se_step(x_ref, w1_ref, b1_ref, w2t_ref, b2_ref, o_ref, *, inv_hw):
    x = x_ref[...]
    s = jnp.sum(x, axis=(1, 2), dtype=jnp.float32) * jnp.float32(inv_hw)
    h = jax.lax.dot_general(s, w1_ref[...], (((1,), (1,)), ((), ())),
                            preferred_element_type=jnp.float32) + b1_ref[...]
    h = h * jax.nn.sigmoid(h)
    g = jnp.dot(h, w2t_ref[...], preferred_element_type=jnp.float32) + b2_ref[...]
    g = jax.nn.sigmoid(g)
    o_ref[...] = x * g[:, None, None, :]


def kernel(x, w1, b1, w2, b2):
    N, C, H, W = x.shape
    R = w1.shape[0]

    xt = jnp.transpose(x, (0, 2, 3, 1))
    w2t = jnp.transpose(w2, (1, 0))

    per_sample = C * H * W * jnp.dtype(x.dtype).itemsize
    nb = 1
    for d in range(1, N + 1):
        if N % d == 0 and N // d >= 4 and 4 * d * per_sample <= (48 << 20):
            nb = d

    out = pl.pallas_call(
        functools.partial(_se_step, inv_hw=1.0 / (H * W)),
        out_shape=jax.ShapeDtypeStruct((N, H, W, C), x.dtype),
        grid=(N // nb,),
        in_specs=[
            pl.BlockSpec((nb, H, W, C), lambda i: (i, 0, 0, 0)),
            pl.BlockSpec((R, C), lambda i: (0, 0)),
            pl.BlockSpec((1, R), lambda i: (0, 0)),
            pl.BlockSpec((R, C), lambda i: (0, 0)),
            pl.BlockSpec((1, C), lambda i: (0, 0)),
        ],
        out_specs=pl.BlockSpec((nb, H, W, C), lambda i: (i, 0, 0, 0)),
        compiler_params=pltpu.CompilerParams(
            dimension_semantics=("parallel",),
            vmem_limit_bytes=int(56 << 20)),
    )(xt, w1, b1.reshape(1, R), w2t, b2.reshape(1, C))

    return jnp.transpose(out, (0, 3, 1, 2))
